# baseline (speedup 1.0000x reference)
import functools

import jax
import jax.numpy as jnp
from jax import lax
from jax.experimental import pallas as pl
from jax.experimental.pallas import tpu as pltpu
from jax.experimental.pallas import tpu_sc as plsc

_EMB = 64
_SCALE = 8.0
_L = 16
_NC = 2
_NS = 16
_NW = _NC * _NS

_IDXW = 128
_CHUNK = 256
_NSTREAM = _CHUNK // _IDXW
_NBUF = 4
_NOBUF = 2
_LOOKAHEAD = 2

_NJ = 200
_NA = _EMB // 8
_NCT = 4096 // _IDXW
_CPC = _CHUNK // _IDXW


@functools.lru_cache(maxsize=None)
def _make_kernel(n_idx):
    b_per_w = n_idx // _NW
    n_chunks = b_per_w // _CHUNK
    idx_rows_per_w = b_per_w // _IDXW
    chunks_per_j = 4096 // _CHUNK

    mesh = plsc.VectorSubcoreMesh(core_axis_name="c", subcore_axis_name="s")

    @functools.partial(
        pl.kernel,
        mesh=mesh,
        compiler_params=pltpu.CompilerParams(use_tc_tiling_on_sc=False,
                                             skip_device_barrier=True,
                                             needs_layout_passes=False),
        out_type=jax.ShapeDtypeStruct((_NJ, _NA, _NCT, 8, _IDXW),
                                      jnp.float32),
        scratch_types=[
            pltpu.VMEM((idx_rows_per_w, _IDXW), jnp.int32),
            pltpu.VMEM((_NBUF, _CHUNK, _EMB), jnp.float32),
            pltpu.VMEM((_NOBUF, _NA, _CPC, 8, _IDXW), jnp.float32),
        ] + [pltpu.SemaphoreType.DMA] * (_NBUF + _NOBUF),
    )
    def emb(x_hbm, table_hbm, out_hbm, idx_all, rows, obuf, *sems):
        gsem = sems[:_NBUF]
        osem = sems[_NBUF:]
        wid = lax.axis_index("s") * _NC + lax.axis_index("c")
        t0 = wid * n_chunks

        pltpu.sync_copy(x_hbm.at[pl.ds(wid * idx_rows_per_w, idx_rows_per_w)],
                        idx_all)

        iot = lax.iota(jnp.int32, 16)

        def start_gather(t, b):
            for j in range(_NSTREAM):
                pltpu.async_copy(
                    table_hbm.at[idx_all.at[t * _NSTREAM + j]],
                    rows.at[b, pl.ds(j * _IDXW, _IDXW)],
                    gsem[b])

        def wait_gather(t, b):
            for j in range(_NSTREAM):
                pltpu.make_async_copy(
                    table_hbm.at[idx_all.at[t * _NSTREAM + j]],
                    rows.at[b, pl.ds(j * _IDXW, _IDXW)],
                    gsem[b]).wait()

        def out_slice(t):
            gt = t0 + t
            j = gt // chunks_per_j
            ib = gt % chunks_per_j
            return out_hbm.at[j, :, pl.ds(_CPC * ib, _CPC)]

        def start_out(t, ob):
            pltpu.async_copy(obuf.at[ob], out_slice(t), osem[ob])

        def wait_out(t, ob):
            pltpu.make_async_copy(obuf.at[ob], out_slice(t),
                                  osem[ob]).wait()

        def assemble(b, ob):
            def a_body(a, carry):
                e0 = a * 8

                def cl_body(cl, carry2):
                    ridx = iot + cl * _L
                    cc = lax.shift_right_logical(cl, 3)
                    lg16 = (cl & 7) * _L
                    for r in range(8):
                        cidx = jnp.full((16,), e0 + r, jnp.int32)
                        v = plsc.load_gather(rows.at[b], [ridx, cidx])
                        obuf[ob, a, cc, r, pl.ds(lg16, _L)] = v * _SCALE
                    return carry2

                lax.fori_loop(0, _CHUNK // _L, cl_body, 0, unroll=False)
                return carry

            lax.fori_loop(0, _NA, a_body, 0, unroll=False)

        def step(t, b, ob, refill, obuf_wait):
            if refill:
                start_gather(t + _LOOKAHEAD, (b + _LOOKAHEAD) % _NBUF)
            if obuf_wait:
                wait_out(t - _NOBUF, ob)
            wait_gather(t, b)
            assemble(b, ob)
            start_out(t, ob)

        for t in range(_LOOKAHEAD):
            start_gather(t, t % _NBUF)

        for t in range(_NBUF):
            step(t, t % _NBUF, t % _NOBUF, refill=True,
                 obuf_wait=(t >= _NOBUF))

        def group_body(g, carry):
            for bi in range(_NBUF):
                step(g * _NBUF + bi, bi, bi % _NOBUF, refill=True,
                     obuf_wait=True)
            return carry

        lax.fori_loop(1, n_chunks // _NBUF - 1, group_body, 0, unroll=False)

        for t in range(n_chunks - _NBUF, n_chunks):
            step(t, t % _NBUF, t % _NOBUF,
                 refill=(t + _LOOKAHEAD < n_chunks), obuf_wait=True)

        for t in range(n_chunks - _NOBUF, n_chunks):
            wait_out(t, t % _NOBUF)

    return emb


def kernel(x, table):
    n_idx = x.size
    xt = x.T.reshape(n_idx // _IDXW, _IDXW).astype(jnp.int32)
    out5 = _make_kernel(n_idx)(xt, table)
    return out5.transpose(2, 4, 0, 1, 3).reshape(4096, _NJ, _EMB)


# speedup vs baseline: 1.0020x; 1.0020x over previous
import functools

import jax
import jax.numpy as jnp
from jax import lax
from jax.experimental import pallas as pl
from jax.experimental.pallas import tpu as pltpu
from jax.experimental.pallas import tpu_sc as plsc

_EMB = 64
_SCALE = 8.0
_L = 16
_NC = 2
_NS = 16
_NW = _NC * _NS

_IDXW = 128
_CHUNK = 256
_NSTREAM = _CHUNK // _IDXW
_NBUF = 4
_NOBUF = 2
_LOOKAHEAD = 2

_NJ = 200
_NA = _EMB // 8
_NCT = 4096 // _IDXW
_CPC = _CHUNK // _IDXW


@functools.lru_cache(maxsize=None)
def _make_kernel(n_idx):
    b_per_w = n_idx // _NW
    n_chunks = b_per_w // _CHUNK
    idx_rows_per_w = b_per_w // _IDXW
    chunks_per_j = 4096 // _CHUNK

    mesh = plsc.VectorSubcoreMesh(core_axis_name="c", subcore_axis_name="s")

    @functools.partial(
        pl.kernel,
        mesh=mesh,
        compiler_params=pltpu.CompilerParams(use_tc_tiling_on_sc=False,
                                             skip_device_barrier=True,
                                             needs_layout_passes=False),
        out_type=jax.ShapeDtypeStruct((_NJ, _NA, _NCT, 8, _IDXW),
                                      jnp.float32),
        scratch_types=[
            pltpu.VMEM((idx_rows_per_w, _IDXW), jnp.int32),
            pltpu.VMEM((_NBUF, _CHUNK, _EMB), jnp.float32),
            pltpu.VMEM((_NOBUF, _NA, _CPC, 8, _IDXW), jnp.float32),
        ] + [pltpu.SemaphoreType.DMA] * (_NBUF + _NOBUF),
    )
    def emb(x_hbm, table_hbm, out_hbm, idx_all, rows, obuf, *sems):
        gsem = sems[:_NBUF]
        osem = sems[_NBUF:]
        wid = lax.axis_index("s") * _NC + lax.axis_index("c")
        t0 = wid * n_chunks

        pltpu.sync_copy(x_hbm.at[pl.ds(wid * idx_rows_per_w, idx_rows_per_w)],
                        idx_all)

        iot = lax.iota(jnp.int32, 16)

        def start_gather(t, b):
            for j in range(_NSTREAM):
                pltpu.async_copy(
                    table_hbm.at[idx_all.at[t * _NSTREAM + j]],
                    rows.at[b, pl.ds(j * _IDXW, _IDXW)],
                    gsem[b])

        def wait_gather(t, b):
            for j in range(_NSTREAM):
                pltpu.make_async_copy(
                    table_hbm.at[idx_all.at[t * _NSTREAM + j]],
                    rows.at[b, pl.ds(j * _IDXW, _IDXW)],
                    gsem[b]).wait()

        def out_slice(t):
            gt = t0 + t
            j = gt // chunks_per_j
            ib = gt % chunks_per_j
            return out_hbm.at[j, :, pl.ds(_CPC * ib, _CPC)]

        def start_out(t, ob):
            pltpu.async_copy(obuf.at[ob], out_slice(t), osem[ob])

        def wait_out(t, ob):
            pltpu.make_async_copy(obuf.at[ob], out_slice(t),
                                  osem[ob]).wait()

        def assemble(b, ob):
            def cl_body(cl, carry):
                ridx = iot + cl * _L
                cc = lax.shift_right_logical(cl, 3)
                lg16 = (cl & 7) * _L
                for a in range(_NA):
                    for r in range(8):
                        cidx = jnp.full((16,), a * 8 + r, jnp.int32)
                        v = plsc.load_gather(rows.at[b], [ridx, cidx])
                        obuf[ob, a, cc, r, pl.ds(lg16, _L)] = v * _SCALE
                return carry

            lax.fori_loop(0, _CHUNK // _L, cl_body, 0, unroll=False)

        def step(t, b, ob):
            rc = t + _LOOKAHEAD

            @pl.when(rc < n_chunks)
            def _():
                start_gather(rc, (b + _LOOKAHEAD) % _NBUF)

            @pl.when(t >= _NOBUF)
            def _():
                wait_out(t - _NOBUF, ob)

            wait_gather(t, b)
            assemble(b, ob)
            start_out(t, ob)

        for t in range(_LOOKAHEAD):
            start_gather(t, t % _NBUF)

        def group_body(g, carry):
            for bi in range(_NBUF):
                step(g * _NBUF + bi, bi, bi % _NOBUF)
            return carry

        lax.fori_loop(0, n_chunks // _NBUF, group_body, 0, unroll=False)

        for t in range(n_chunks - _NOBUF, n_chunks):
            wait_out(t, t % _NOBUF)

    return emb


def kernel(x, table):
    n_idx = x.size
    xt = x.T.reshape(n_idx // _IDXW, _IDXW).astype(jnp.int32)
    out5 = _make_kernel(n_idx)(xt, table)
    return out5.transpose(2, 4, 0, 1, 3).reshape(4096, _NJ, _EMB)


# speedup vs baseline: 1.7242x; 1.7208x over previous
import functools

import jax
import jax.numpy as jnp
from jax import lax
from jax.experimental import pallas as pl
from jax.experimental.pallas import tpu as pltpu
from jax.experimental.pallas import tpu_sc as plsc

_EMB = 64
_SCALE = 8.0
_L = 16
_NC = 2
_NS = 16
_NW = _NC * _NS

_IDXW = 128
_CHUNK = 256
_NSTREAM = _CHUNK // _IDXW
_NBUF = 4
_NOBUF = 2
_LOOKAHEAD = 2

_NJ = 200
_NA = _EMB // 8
_NCT = 4096 // _IDXW
_CPC = _CHUNK // _IDXW


@functools.lru_cache(maxsize=None)
def _make_kernel(n_idx):
    b_per_w = n_idx // _NW
    n_chunks = b_per_w // _CHUNK
    idx_rows_per_w = b_per_w // _IDXW
    chunks_per_j = 4096 // _CHUNK

    mesh = plsc.VectorSubcoreMesh(core_axis_name="c", subcore_axis_name="s")

    @functools.partial(
        pl.kernel,
        mesh=mesh,
        compiler_params=pltpu.CompilerParams(use_tc_tiling_on_sc=False,
                                             skip_device_barrier=True,
                                             needs_layout_passes=False),
        out_type=jax.ShapeDtypeStruct((_NJ, _NA, _NCT * 8, _IDXW),
                                      jnp.float32),
        scratch_types=[
            pltpu.VMEM((idx_rows_per_w, _IDXW), jnp.int32),
            pltpu.VMEM((_NBUF, _CHUNK, _EMB), jnp.float32),
            pltpu.VMEM((_NOBUF * _NA * _CPC * 8, _IDXW + 1), jnp.float32),
        ] + [pltpu.SemaphoreType.DMA] * (_NBUF + _NOBUF),
    )
    def emb(x_hbm, table_hbm, out_hbm, idx_all, rows, obuf, *sems):
        gsem = sems[:_NBUF]
        osem = sems[_NBUF:]
        wid = lax.axis_index("s") * _NC + lax.axis_index("c")
        t0 = wid * n_chunks

        pltpu.sync_copy(x_hbm.at[pl.ds(wid * idx_rows_per_w, idx_rows_per_w)],
                        idx_all)

        iot = lax.iota(jnp.int32, 16)

        def start_gather(t, b):
            for j in range(_NSTREAM):
                pltpu.async_copy(
                    table_hbm.at[idx_all.at[t * _NSTREAM + j]],
                    rows.at[b, pl.ds(j * _IDXW, _IDXW)],
                    gsem[b])

        def wait_gather(t, b):
            for j in range(_NSTREAM):
                pltpu.make_async_copy(
                    table_hbm.at[idx_all.at[t * _NSTREAM + j]],
                    rows.at[b, pl.ds(j * _IDXW, _IDXW)],
                    gsem[b]).wait()

        def out_copies(t, ob):
            gt = t0 + t
            j = gt // chunks_per_j
            ib = gt % chunks_per_j
            nseg = _CPC * 8
            for a in range(_NA):
                src = obuf.at[pl.ds((ob * _NA + a) * nseg, nseg),
                              pl.ds(0, _IDXW)]
                dst = out_hbm.at[j, a, pl.ds(nseg * ib, nseg)]
                yield src, dst

        def start_out(t, ob):
            for src, dst in out_copies(t, ob):
                pltpu.async_copy(src, dst, osem[ob])

        def wait_out(t, ob):
            for src, dst in out_copies(t, ob):
                pltpu.make_async_copy(src, dst, osem[ob]).wait()

        rowpat = [(lax.shift_right_logical(iot + e0, 3) * (_CPC * 8)
                   + ((iot + e0) & 7)) for e0 in range(0, _EMB, _L)]

        def assemble(b, ob):
            base0 = ob * (_NA * _CPC * 8)

            def i_body(i, carry):
                cc8 = base0 + lax.shift_right_logical(i, 7) * 8
                rsplat = jnp.full((16,), cc8, jnp.int32)
                csplat = jnp.full((16,), i & (_IDXW - 1), jnp.int32)
                for g in range(_EMB // _L):
                    v = rows[b, i, pl.ds(g * _L, _L)] * _SCALE
                    plsc.store_scatter(obuf, [rowpat[g] + rsplat, csplat], v)
                return carry

            lax.fori_loop(0, _CHUNK, i_body, 0, unroll=4)

        def step(t, b, ob):
            rc = t + _LOOKAHEAD

            @pl.when(rc < n_chunks)
            def _():
                start_gather(rc, (b + _LOOKAHEAD) % _NBUF)

            @pl.when(t >= _NOBUF)
            def _():
                wait_out(t - _NOBUF, ob)

            wait_gather(t, b)
            assemble(b, ob)
            start_out(t, ob)

        for t in range(_LOOKAHEAD):
            start_gather(t, t % _NBUF)

        def group_body(g, carry):
            for bi in range(_NBUF):
                step(g * _NBUF + bi, bi, bi % _NOBUF)
            return carry

        lax.fori_loop(0, n_chunks // _NBUF, group_body, 0, unroll=False)

        for t in range(n_chunks - _NOBUF, n_chunks):
            wait_out(t, t % _NOBUF)

    return emb


def kernel(x, table):
    n_idx = x.size
    xt = x.T.reshape(n_idx // _IDXW, _IDXW).astype(jnp.int32)
    out4 = _make_kernel(n_idx)(xt, table)
    out5 = out4.reshape(_NJ, _NA, _NCT, 8, _IDXW)
    return out5.transpose(2, 4, 0, 1, 3).reshape(4096, _NJ, _EMB)


# speedup vs baseline: 2.5818x; 1.4974x over previous
import functools

import jax
import jax.numpy as jnp
from jax import lax
from jax.experimental import pallas as pl
from jax.experimental.pallas import tpu as pltpu
from jax.experimental.pallas import tpu_sc as plsc

_EMB = 64
_SCALE = 8.0
_L = 16
_NC = 2
_NS = 16
_NW = _NC * _NS

_IDXW = 128
_CHUNK = 256
_NSTREAM = _CHUNK // _IDXW
_NBUF = 4
_NOBUF = 2
_LOOKAHEAD = 2

_NJ = 200
_NA = _EMB // 8
_NCT = 4096 // _IDXW
_CPC = _CHUNK // _IDXW


@functools.lru_cache(maxsize=None)
def _make_kernel(n_idx):
    b_per_w = n_idx // _NW
    n_chunks = b_per_w // _CHUNK
    idx_rows_per_w = b_per_w // _IDXW
    chunks_per_j = 4096 // _CHUNK

    mesh = plsc.VectorSubcoreMesh(core_axis_name="c", subcore_axis_name="s")

    @functools.partial(
        pl.kernel,
        mesh=mesh,
        compiler_params=pltpu.CompilerParams(use_tc_tiling_on_sc=False,
                                             skip_device_barrier=True,
                                             needs_layout_passes=False),
        out_type=jax.ShapeDtypeStruct((_NJ, _NA, _NCT * 8, _IDXW),
                                      jnp.float32),
        scratch_types=[
            pltpu.VMEM((idx_rows_per_w, _IDXW), jnp.int32),
            pltpu.VMEM((_NBUF, _CHUNK, _EMB), jnp.float32),
            pltpu.VMEM((_NOBUF * _NA * _CPC * 8, _IDXW + 1), jnp.float32),
        ] + [pltpu.SemaphoreType.DMA] * (_NBUF + _NOBUF),
    )
    def emb(x_hbm, table_hbm, out_hbm, idx_all, rows, obuf, *sems):
        gsem = sems[:_NBUF]
        osem = sems[_NBUF:]
        wid = lax.axis_index("s") * _NC + lax.axis_index("c")
        t0 = wid * n_chunks

        pltpu.sync_copy(x_hbm.at[pl.ds(wid * idx_rows_per_w, idx_rows_per_w)],
                        idx_all)

        iot = lax.iota(jnp.int32, 16)

        def start_gather(t, b):
            for j in range(_NSTREAM):
                pltpu.async_copy(
                    table_hbm.at[idx_all.at[t * _NSTREAM + j]],
                    rows.at[b, pl.ds(j * _IDXW, _IDXW)],
                    gsem[b])

        def wait_gather(t, b):
            for j in range(_NSTREAM):
                pltpu.make_async_copy(
                    table_hbm.at[idx_all.at[t * _NSTREAM + j]],
                    rows.at[b, pl.ds(j * _IDXW, _IDXW)],
                    gsem[b]).wait()

        def out_copies(t, ob):
            gt = t0 + t
            j = gt // chunks_per_j
            ib = gt % chunks_per_j
            nseg = _CPC * 8
            for a in range(_NA):
                src = obuf.at[pl.ds((ob * _NA + a) * nseg, nseg),
                              pl.ds(0, _IDXW)]
                dst = out_hbm.at[j, a, pl.ds(nseg * ib, nseg)]
                yield src, dst

        def start_out(t, ob):
            for src, dst in out_copies(t, ob):
                pltpu.async_copy(src, dst, osem[ob])

        def wait_out(t, ob):
            for src, dst in out_copies(t, ob):
                pltpu.make_async_copy(src, dst, osem[ob]).wait()

        rowpat = [(lax.shift_right_logical(iot + e0, 3) * (_CPC * 8)
                   + ((iot + e0) & 7)) for e0 in range(0, _EMB, _L)]

        def assemble(b, ob):
            base0 = ob * (_NA * _CPC * 8)

            def i_body(i):
                cc8 = base0 + lax.shift_right_logical(i, 7) * 8
                rsplat = jnp.full((16,), cc8, jnp.int32)
                csplat = jnp.full((16,), i & (_IDXW - 1), jnp.int32)
                for g in range(_EMB // _L):
                    v = rows[b, i, pl.ds(g * _L, _L)] * _SCALE
                    plsc.store_scatter(obuf, [rowpat[g] + rsplat, csplat], v)

            plsc.parallel_loop(0, _CHUNK, 1, unroll=8)(i_body)

        def step(t, b, ob):
            rc = t + _LOOKAHEAD

            @pl.when(rc < n_chunks)
            def _():
                start_gather(rc, (b + _LOOKAHEAD) % _NBUF)

            @pl.when(t >= _NOBUF)
            def _():
                wait_out(t - _NOBUF, ob)

            wait_gather(t, b)
            assemble(b, ob)
            start_out(t, ob)

        for t in range(_LOOKAHEAD):
            start_gather(t, t % _NBUF)

        def group_body(g, carry):
            for bi in range(_NBUF):
                step(g * _NBUF + bi, bi, bi % _NOBUF)
            return carry

        lax.fori_loop(0, n_chunks // _NBUF, group_body, 0, unroll=False)

        for t in range(n_chunks - _NOBUF, n_chunks):
            wait_out(t, t % _NOBUF)

    return emb


def kernel(x, table):
    n_idx = x.size
    xt = x.T.reshape(n_idx // _IDXW, _IDXW).astype(jnp.int32)
    out4 = _make_kernel(n_idx)(xt, table)
    out5 = out4.reshape(_NJ, _NA, _NCT, 8, _IDXW)
    return out5.transpose(2, 4, 0, 1, 3).reshape(4096, _NJ, _EMB)
